# Initial kernel scaffold; baseline (speedup 1.0000x reference)
import jax
import jax.numpy as jnp
from jax.experimental import pallas as pl


def kernel(node_feat_table, nodes_id, W_init, b_init, W_msg, b_msg, W_upd, b_upd, W_dec, b_dec, edge_index, edge_attr, root_ids):
    raise NotImplementedError("write your pallas kernel here")


# speedup vs baseline: 1.2169x; 1.2169x over previous
import functools

import jax
import jax.numpy as jnp
from jax import lax
from jax.experimental import pallas as pl
from jax.experimental.pallas import tpu as pltpu
from jax.experimental.pallas import tpu_sc as plsc

N_NODES = 10000
N_EDGES = 160000
HID = 512
NC = 2
NS = 16
NW = NC * NS
NPASS = 4
WROWS = 80
N_PAD = NPASS * NW * WROWS
NBOUND = 136
EB = 64
E_PADDED = N_EDGES + 3840
HSL = HID // 16

_MESH = plsc.VectorSubcoreMesh(core_axis_name="c", subcore_axis_name="s")


def _pre_body(table, nid, ori, nidv, rows, sem):
    c = lax.axis_index("c")
    s = lax.axis_index("s")
    wid = s * NC + c
    start = jnp.minimum(wid * 320, N_NODES - 320)
    for b in range(5):
        off = start + b * 64
        pltpu.sync_copy(nid.at[pl.ds(off, 64)], nidv)
        pltpu.async_copy(table.at[nidv], rows, sem).wait()
        pltpu.sync_copy(rows, ori.at[pl.ds(off, 64)])


def _pre_gather(table, nid):
    k = pl.kernel(
        _pre_body,
        out_type=jax.ShapeDtypeStruct((N_NODES, 256), jnp.float32),
        mesh=_MESH,
        compiler_params=pltpu.CompilerParams(needs_layout_passes=False),
        scratch_types=[
            pltpu.VMEM((64,), jnp.int32),
            pltpu.VMEM((64, 256), jnp.float32),
            pltpu.SemaphoreType.DMA,
        ],
    )
    return k(table, nid)


def _hop_body(p_hbm, e_hbm, src_hbm, dst_hbm, oid_hbm, bounds_hbm, agg_hbm,
              accum, prow, erow, srcv, dstv, oidv, boundv, sem):
    c = lax.axis_index("c")
    s = lax.axis_index("s")
    wid = s * NC + c
    lane = lax.broadcasted_iota(jnp.int32, (16,), 0)
    zero16 = jnp.zeros((16,), jnp.float32)
    for p in range(NPASS):
        w = p * NW + wid
        my_base = w * WROWS

        def zfn(r, z):
            for kk in range(HSL):
                accum[r, pl.ds(kk * 16, 16)] = zero16
            return z

        lax.fori_loop(0, WROWS, zfn, 0)
        q = (w // 8) * 8
        pltpu.sync_copy(bounds_hbm.at[pl.ds(q, 16)], boundv)
        lo = plsc.load_gather(
            boundv, [jnp.full((16,), w - q, jnp.int32)])[0]
        hi = plsc.load_gather(
            boundv, [jnp.full((16,), w - q + 1, jnp.int32)])[0]
        e0 = (lo // 8) * 8
        nb = (hi - e0 + EB - 1) // EB

        def batch(b, z):
            off = e0 + b * EB
            pltpu.sync_copy(dst_hbm.at[pl.ds(off, EB)], dstv)
            pltpu.sync_copy(src_hbm.at[pl.ds(off, EB)], srcv)
            pltpu.sync_copy(oid_hbm.at[pl.ds(off, EB)], oidv)
            cp1 = pltpu.async_copy(p_hbm.at[srcv], prow, sem)
            cp2 = pltpu.async_copy(e_hbm.at[oidv], erow, sem)
            cp1.wait()
            cp2.wait()

            def rowfn(r, z2):
                r16 = jnp.full((16,), r, jnp.int32)
                d = plsc.load_gather(dstv, [r16])
                loc = d - my_base
                m = (loc >= 0) & (loc < WROWS)
                for kk in range(HSL):
                    v = (prow[r, pl.ds(kk * 16, 16)]
                         + erow[r, pl.ds(kk * 16, 16)])
                    v = jnp.maximum(v, 0.0)
                    plsc.addupdate_scatter(
                        accum, [loc, kk * 16 + lane], v, mask=m)
                return z2

            lax.fori_loop(0, EB, rowfn, 0)
            return z

        lax.fori_loop(0, nb, batch, 0)
        pltpu.sync_copy(accum, agg_hbm.at[pl.ds(my_base, WROWS)])


def _hop(p, e, src_s, dst_s, order, bounds):
    k = pl.kernel(
        _hop_body,
        out_type=jax.ShapeDtypeStruct((N_PAD, HID), jnp.float32),
        mesh=_MESH,
        compiler_params=pltpu.CompilerParams(needs_layout_passes=False),
        scratch_types=[
            pltpu.VMEM((WROWS, HID), jnp.float32),
            pltpu.VMEM((EB, HID), jnp.float32),
            pltpu.VMEM((EB, HID), jnp.float32),
            pltpu.VMEM((EB,), jnp.int32),
            pltpu.VMEM((EB,), jnp.int32),
            pltpu.VMEM((EB,), jnp.int32),
            pltpu.VMEM((16,), jnp.int32),
            pltpu.SemaphoreType.DMA,
        ],
    )
    return k(p, e, src_s, dst_s, order, bounds)


def _dec_body(d_hbm, rid_hbm, out_hbm, ridv, rows, outv, sem):
    c = lax.axis_index("c")
    s = lax.axis_index("s")
    wid = s * NC + c
    pltpu.sync_copy(rid_hbm.at[pl.ds(wid * 64, 64)], ridv)
    pltpu.async_copy(d_hbm.at[ridv], rows, sem).wait()
    lane = lax.broadcasted_iota(jnp.int32, (16,), 0)
    col0 = jnp.zeros((16,), jnp.int32)
    for g in range(2):
        i0 = 2 * lane + g * 32
        a = plsc.load_gather(rows, [i0, col0])
        b = plsc.load_gather(rows, [i0 + 1, col0 + 1])
        outv[pl.ds(g * 16, 16)] = a + b
    pltpu.sync_copy(outv, out_hbm.at[pl.ds(wid * 32, 32)])


def _decode(d, rid):
    n_links = rid.shape[0] // 2
    k = pl.kernel(
        _dec_body,
        out_type=jax.ShapeDtypeStruct((n_links,), jnp.float32),
        mesh=_MESH,
        compiler_params=pltpu.CompilerParams(needs_layout_passes=False),
        scratch_types=[
            pltpu.VMEM((64,), jnp.int32),
            pltpu.VMEM((64, 128), jnp.float32),
            pltpu.VMEM((32,), jnp.float32),
            pltpu.SemaphoreType.DMA,
        ],
    )
    return k(d, rid)


def _emm_kern(x_ref, w_ref, b_ref, o_ref):
    o_ref[...] = (jnp.dot(x_ref[...], w_ref[...],
                          preferred_element_type=jnp.float32) + b_ref[...])


def _edge_mm(x, w, b):
    m, kdim = x.shape
    n = w.shape[1]
    bm = 2000
    return pl.pallas_call(
        _emm_kern,
        grid=(m // bm,),
        in_specs=[
            pl.BlockSpec((bm, kdim), lambda i: (i, 0)),
            pl.BlockSpec((kdim, n), lambda i: (0, 0)),
            pl.BlockSpec((1, n), lambda i: (0, 0)),
        ],
        out_specs=pl.BlockSpec((bm, n), lambda i: (i, 0)),
        out_shape=jax.ShapeDtypeStruct((m, n), jnp.float32),
    )(x, w, b.reshape(1, n))


def _init_kern(x_ref, wi_ref, bi_ref, wm_ref, h_ref, p_ref):
    h = (jnp.dot(x_ref[...], wi_ref[...],
                 preferred_element_type=jnp.float32) + bi_ref[...])
    h_ref[...] = h
    p_ref[...] = jnp.dot(h, wm_ref[...], preferred_element_type=jnp.float32)


def _init_mm(x, wi, bi, wm):
    m, kdim = x.shape
    n = wi.shape[1]
    bm = 1000
    return pl.pallas_call(
        _init_kern,
        grid=(m // bm,),
        in_specs=[
            pl.BlockSpec((bm, kdim), lambda i: (i, 0)),
            pl.BlockSpec((kdim, n), lambda i: (0, 0)),
            pl.BlockSpec((1, n), lambda i: (0, 0)),
            pl.BlockSpec((n, n), lambda i: (0, 0)),
        ],
        out_specs=[
            pl.BlockSpec((bm, n), lambda i: (i, 0)),
            pl.BlockSpec((bm, n), lambda i: (i, 0)),
        ],
        out_shape=[
            jax.ShapeDtypeStruct((m, n), jnp.float32),
            jax.ShapeDtypeStruct((m, n), jnp.float32),
        ],
    )(x, wi, bi.reshape(1, n), wm)


def _upd_kern(h_ref, a_ref, wh_ref, wa_ref, b_ref, wn_ref, h2_ref, p2_ref):
    y = (jnp.dot(h_ref[...], wh_ref[...], preferred_element_type=jnp.float32)
         + jnp.dot(a_ref[...], wa_ref[...], preferred_element_type=jnp.float32)
         + b_ref[...])
    y = jnp.maximum(y, 0.0)
    h2_ref[...] = y
    p2_ref[...] = jnp.dot(y, wn_ref[...], preferred_element_type=jnp.float32)


def _upd_mm(h, a, wh, wa, b, wn):
    m, n = h.shape
    n2 = wn.shape[1]
    bm = 1000
    return pl.pallas_call(
        _upd_kern,
        grid=(m // bm,),
        in_specs=[
            pl.BlockSpec((bm, n), lambda i: (i, 0)),
            pl.BlockSpec((bm, n), lambda i: (i, 0)),
            pl.BlockSpec((n, n), lambda i: (0, 0)),
            pl.BlockSpec((n, n), lambda i: (0, 0)),
            pl.BlockSpec((1, n), lambda i: (0, 0)),
            pl.BlockSpec((n, n2), lambda i: (0, 0)),
        ],
        out_specs=[
            pl.BlockSpec((bm, n), lambda i: (i, 0)),
            pl.BlockSpec((bm, n2), lambda i: (i, 0)),
        ],
        out_shape=[
            jax.ShapeDtypeStruct((m, n), jnp.float32),
            jax.ShapeDtypeStruct((m, n2), jnp.float32),
        ],
    )(h, a, wh, wa, b.reshape(1, n), wn)


def _upd_dec_kern(h_ref, a_ref, wh_ref, wa_ref, b_ref, wd_ref, bd_ref, d_ref):
    y = (jnp.dot(h_ref[...], wh_ref[...], preferred_element_type=jnp.float32)
         + jnp.dot(a_ref[...], wa_ref[...], preferred_element_type=jnp.float32)
         + b_ref[...])
    y = jnp.maximum(y, 0.0)
    d_ref[...] = (jnp.dot(y, wd_ref[...], preferred_element_type=jnp.float32)
                  + bd_ref[...])


def _upd_dec_mm(h, a, wh, wa, b, wd, bd):
    m, n = h.shape
    n2 = wd.shape[1]
    bm = 1000
    return pl.pallas_call(
        _upd_dec_kern,
        grid=(m // bm,),
        in_specs=[
            pl.BlockSpec((bm, n), lambda i: (i, 0)),
            pl.BlockSpec((bm, n), lambda i: (i, 0)),
            pl.BlockSpec((n, n), lambda i: (0, 0)),
            pl.BlockSpec((n, n), lambda i: (0, 0)),
            pl.BlockSpec((1, n), lambda i: (0, 0)),
            pl.BlockSpec((n, n2), lambda i: (0, 0)),
            pl.BlockSpec((1, n2), lambda i: (0, 0)),
        ],
        out_specs=pl.BlockSpec((bm, n2), lambda i: (i, 0)),
        out_shape=jax.ShapeDtypeStruct((m, n2), jnp.float32),
    )(h, a, wh, wa, b.reshape(1, n), wd, bd.reshape(1, n2))


def kernel(node_feat_table, nodes_id, W_init, b_init, W_msg, b_msg,
           W_upd, b_upd, W_dec, b_dec, edge_index, edge_attr, root_ids):
    src = edge_index[0]
    dst = edge_index[1]
    order = jnp.argsort(dst)
    src_s = jnp.take(src, order)
    dst_s = jnp.take(dst, order)
    wm_h = W_msg[:HID]
    wm_e = W_msg[HID:]
    wu_h = W_upd[:HID]
    wu_a = W_upd[HID:]
    wd = jnp.zeros((HID, 128), jnp.float32)
    wd = wd.at[:, 0].set(W_dec[:HID, 0]).at[:, 1].set(W_dec[HID:, 0])
    bd = jnp.zeros((128,), jnp.float32).at[0].set(b_dec[0])
    rid = jnp.stack([root_ids[:, 0], root_ids[:, 1]], axis=1).reshape(-1)
    targets = jnp.arange(NPASS * NW + 1, dtype=jnp.int32) * WROWS
    bounds = jnp.searchsorted(dst_s, targets, side="left").astype(jnp.int32)
    bounds = jnp.concatenate(
        [bounds, jnp.full((NBOUND - bounds.shape[0],), N_EDGES, jnp.int32)])
    npad = E_PADDED - N_EDGES
    src_s = jnp.concatenate([src_s, jnp.zeros((npad,), jnp.int32)])
    dst_s = jnp.concatenate(
        [dst_s, jnp.full((npad,), jnp.int32(0x3FFFFFFF))])
    oid = jnp.concatenate([order, jnp.zeros((npad,), jnp.int32)])

    ori = _pre_gather(node_feat_table, nodes_id)
    e_all = _edge_mm(edge_attr, wm_e, b_msg)
    h0, p1 = _init_mm(ori, W_init, b_init, wm_h)
    agg1 = _hop(p1, e_all, src_s, dst_s, oid, bounds)[:N_NODES]
    h1, p2 = _upd_mm(h0, agg1, wu_h, wu_a, b_upd, wm_h)
    agg2 = _hop(p2, e_all, src_s, dst_s, oid, bounds)[:N_NODES]
    d = _upd_dec_mm(h1, agg2, wu_h, wu_a, b_upd, wd, bd)
    out = _decode(d, rid)
    return out.reshape(-1, 1)


# speedup vs baseline: 2.0103x; 1.6519x over previous
import jax
import jax.numpy as jnp
from jax import lax
from jax.experimental import pallas as pl
from jax.experimental.pallas import tpu as pltpu
from jax.experimental.pallas import tpu_sc as plsc

N_NODES = 10000
N_EDGES = 160000
HID = 512
CB = 128
NCB = HID // CB
NC = 2
NS = 16
EPT = N_EDGES // NS
EB = 80
NB = EPT // EB
RPT = 624
ZR = 208
HSL = CB // 16

_MESH = plsc.VectorSubcoreMesh(core_axis_name="c", subcore_axis_name="s")
_SC_PARAMS = pltpu.CompilerParams(needs_layout_passes=False)


def _pre_body(table, nid, ori, nidv, rows, sem):
    c = lax.axis_index("c")
    s = lax.axis_index("s")
    wid = s * NC + c
    start = jnp.minimum(wid * 320, N_NODES - 320)
    for b in range(5):
        off = start + b * 64
        pltpu.sync_copy(nid.at[pl.ds(off, 64)], nidv)
        pltpu.async_copy(table.at[nidv], rows, sem).wait()
        pltpu.sync_copy(rows, ori.at[pl.ds(off, 64)])


def _pre_gather(table, nid):
    k = pl.kernel(
        _pre_body,
        out_type=jax.ShapeDtypeStruct((N_NODES, 256), jnp.float32),
        mesh=_MESH,
        compiler_params=_SC_PARAMS,
        scratch_types=[
            pltpu.VMEM((64,), jnp.int32),
            pltpu.VMEM((64, 256), jnp.float32),
            pltpu.SemaphoreType.DMA,
        ],
    )
    return k(table, nid)


def _hop_body(p0, p1, p2, p3, e0, e1, e2, e3, src_hbm, dst_hbm,
              a0, a1, a2, a3,
              shared, prow, erow, srcv, dst2d, zbuf, sem):
    c = lax.axis_index("c")
    s = lax.axis_index("s")
    zero16 = jnp.zeros((16,), jnp.float32)

    def zb(r, z):
        for k in range(HSL):
            zbuf[r, pl.ds(k * 16, 16)] = zero16
        return z

    lax.fori_loop(0, ZR, zb, 0)
    ebase = s * EPT

    def do_block(pq, eq, aq):
        for j in range(3):
            pltpu.sync_copy(zbuf, shared.at[pl.ds(s * RPT + j * ZR, ZR)])

        @pl.when(s == NS - 1)
        def _():
            pltpu.sync_copy(zbuf.at[pl.ds(0, 16)],
                            shared.at[pl.ds(NS * RPT, 16)])

        plsc.subcore_barrier()

        def batch(b, z):
            off = ebase + b * EB
            pltpu.sync_copy(src_hbm.at[pl.ds(off, EB)], srcv)
            pltpu.sync_copy(dst_hbm.at[pl.ds(off, EB)], dst2d.at[0])
            cp = pltpu.async_copy(pq.at[srcv], prow, sem)
            pltpu.sync_copy(eq.at[pl.ds(off, EB)], erow)
            cp.wait()

            def rfn(r, z2):
                for k in range(HSL):
                    sl = pl.ds(k * 16, 16)
                    prow[r, sl] = jnp.maximum(prow[r, sl] + erow[r, sl], 0.0)
                return z2

            lax.fori_loop(0, EB, rfn, 0)
            pltpu.sync_copy(prow, shared.at[dst2d.at[0]], add=True)
            return z

        lax.fori_loop(0, NB, batch, 0)
        plsc.subcore_barrier()

        pltpu.sync_copy(shared.at[pl.ds(s * RPT, RPT)],
                        aq.at[pl.ds(s * RPT, RPT)])

        @pl.when(s == NS - 1)
        def _():
            pltpu.sync_copy(shared.at[pl.ds(NS * RPT, 16)],
                            aq.at[pl.ds(NS * RPT, 16)])

        plsc.subcore_barrier()

    @pl.when(c == 0)
    def _():
        do_block(p0, e0, a0)
        do_block(p1, e1, a1)

    @pl.when(c == 1)
    def _():
        do_block(p2, e2, a2)
        do_block(p3, e3, a3)


def _hop(pblks, eblks, src, dst):
    k = pl.kernel(
        _hop_body,
        out_type=[jax.ShapeDtypeStruct((N_NODES, CB), jnp.float32)] * NCB,
        mesh=_MESH,
        compiler_params=_SC_PARAMS,
        scratch_types=[
            pltpu.VMEM_SHARED((N_NODES, CB), jnp.float32),
            pltpu.VMEM((EB, CB), jnp.float32),
            pltpu.VMEM((EB, CB), jnp.float32),
            pltpu.VMEM((EB,), jnp.int32),
            pltpu.VMEM((1, EB), jnp.int32),
            pltpu.VMEM((ZR, CB), jnp.float32),
            pltpu.SemaphoreType.DMA,
        ],
    )
    return k(*pblks, *eblks, src, dst)


def _dec_body(d_hbm, rid_hbm, out_hbm, ridv, rows, outv, sem):
    c = lax.axis_index("c")
    s = lax.axis_index("s")
    wid = s * NC + c
    pltpu.sync_copy(rid_hbm.at[pl.ds(wid * 64, 64)], ridv)
    pltpu.async_copy(d_hbm.at[ridv], rows, sem).wait()
    lane = lax.broadcasted_iota(jnp.int32, (16,), 0)
    col0 = jnp.zeros((16,), jnp.int32)
    for g in range(2):
        i0 = 2 * lane + g * 32
        a = plsc.load_gather(rows, [i0, col0])
        b = plsc.load_gather(rows, [i0 + 1, col0 + 1])
        outv[pl.ds(g * 16, 16)] = a + b
    pltpu.sync_copy(outv, out_hbm.at[pl.ds(wid * 32, 32)])


def _decode(d, rid):
    n_links = rid.shape[0] // 2
    k = pl.kernel(
        _dec_body,
        out_type=jax.ShapeDtypeStruct((n_links,), jnp.float32),
        mesh=_MESH,
        compiler_params=_SC_PARAMS,
        scratch_types=[
            pltpu.VMEM((64,), jnp.int32),
            pltpu.VMEM((64, 128), jnp.float32),
            pltpu.VMEM((32,), jnp.float32),
            pltpu.SemaphoreType.DMA,
        ],
    )
    return k(d, rid)


def _emm_kern(x_ref, w_ref, b_ref, o0, o1, o2, o3):
    o = (jnp.dot(x_ref[...], w_ref[...],
                 preferred_element_type=jnp.float32) + b_ref[...])
    for q, oq in enumerate((o0, o1, o2, o3)):
        oq[...] = o[:, q * CB:(q + 1) * CB]


def _edge_mm(x, w, b):
    m, kdim = x.shape
    n = w.shape[1]
    bm = 2000
    return pl.pallas_call(
        _emm_kern,
        grid=(m // bm,),
        in_specs=[
            pl.BlockSpec((bm, kdim), lambda i: (i, 0)),
            pl.BlockSpec((kdim, n), lambda i: (0, 0)),
            pl.BlockSpec((1, n), lambda i: (0, 0)),
        ],
        out_specs=[pl.BlockSpec((bm, CB), lambda i: (i, 0))] * NCB,
        out_shape=[jax.ShapeDtypeStruct((m, CB), jnp.float32)] * NCB,
    )(x, w, b.reshape(1, n))


def _init_kern(x_ref, wi_ref, bi_ref, wm_ref, h_ref, p0, p1, p2, p3):
    h = (jnp.dot(x_ref[...], wi_ref[...],
                 preferred_element_type=jnp.float32) + bi_ref[...])
    h_ref[...] = h
    for q, pq in enumerate((p0, p1, p2, p3)):
        pq[...] = jnp.dot(h, wm_ref[:, q * CB:(q + 1) * CB],
                          preferred_element_type=jnp.float32)


def _init_mm(x, wi, bi, wm):
    m, kdim = x.shape
    n = wi.shape[1]
    bm = 1000
    return pl.pallas_call(
        _init_kern,
        grid=(m // bm,),
        in_specs=[
            pl.BlockSpec((bm, kdim), lambda i: (i, 0)),
            pl.BlockSpec((kdim, n), lambda i: (0, 0)),
            pl.BlockSpec((1, n), lambda i: (0, 0)),
            pl.BlockSpec((n, n), lambda i: (0, 0)),
        ],
        out_specs=[pl.BlockSpec((bm, n), lambda i: (i, 0))]
        + [pl.BlockSpec((bm, CB), lambda i: (i, 0))] * NCB,
        out_shape=[jax.ShapeDtypeStruct((m, n), jnp.float32)]
        + [jax.ShapeDtypeStruct((m, CB), jnp.float32)] * NCB,
    )(x, wi, bi.reshape(1, n), wm)


def _upd_kern(h_ref, a0, a1, a2, a3, wh_ref, wa_ref, b_ref, wn_ref,
              h2_ref, p0, p1, p2, p3):
    y = (jnp.dot(h_ref[...], wh_ref[...], preferred_element_type=jnp.float32)
         + b_ref[...])
    for q, aq in enumerate((a0, a1, a2, a3)):
        y = y + jnp.dot(aq[...], wa_ref[q * CB:(q + 1) * CB, :],
                        preferred_element_type=jnp.float32)
    y = jnp.maximum(y, 0.0)
    h2_ref[...] = y
    for q, pq in enumerate((p0, p1, p2, p3)):
        pq[...] = jnp.dot(y, wn_ref[:, q * CB:(q + 1) * CB],
                          preferred_element_type=jnp.float32)


def _upd_mm(h, ablks, wh, wa, b, wn):
    m, n = h.shape
    bm = 1000
    return pl.pallas_call(
        _upd_kern,
        grid=(m // bm,),
        in_specs=[pl.BlockSpec((bm, n), lambda i: (i, 0))]
        + [pl.BlockSpec((bm, CB), lambda i: (i, 0))] * NCB
        + [
            pl.BlockSpec((n, n), lambda i: (0, 0)),
            pl.BlockSpec((n, n), lambda i: (0, 0)),
            pl.BlockSpec((1, n), lambda i: (0, 0)),
            pl.BlockSpec((n, n), lambda i: (0, 0)),
        ],
        out_specs=[pl.BlockSpec((bm, n), lambda i: (i, 0))]
        + [pl.BlockSpec((bm, CB), lambda i: (i, 0))] * NCB,
        out_shape=[jax.ShapeDtypeStruct((m, n), jnp.float32)]
        + [jax.ShapeDtypeStruct((m, CB), jnp.float32)] * NCB,
    )(h, *ablks, wh, wa, b.reshape(1, n), wn)


def _upd_dec_kern(h_ref, a0, a1, a2, a3, wh_ref, wa_ref, b_ref,
                  wd_ref, bd_ref, d_ref):
    y = (jnp.dot(h_ref[...], wh_ref[...], preferred_element_type=jnp.float32)
         + b_ref[...])
    for q, aq in enumerate((a0, a1, a2, a3)):
        y = y + jnp.dot(aq[...], wa_ref[q * CB:(q + 1) * CB, :],
                        preferred_element_type=jnp.float32)
    y = jnp.maximum(y, 0.0)
    d_ref[...] = (jnp.dot(y, wd_ref[...], preferred_element_type=jnp.float32)
                  + bd_ref[...])


def _upd_dec_mm(h, ablks, wh, wa, b, wd, bd):
    m, n = h.shape
    n2 = wd.shape[1]
    bm = 1000
    return pl.pallas_call(
        _upd_dec_kern,
        grid=(m // bm,),
        in_specs=[pl.BlockSpec((bm, n), lambda i: (i, 0))]
        + [pl.BlockSpec((bm, CB), lambda i: (i, 0))] * NCB
        + [
            pl.BlockSpec((n, n), lambda i: (0, 0)),
            pl.BlockSpec((n, n), lambda i: (0, 0)),
            pl.BlockSpec((1, n), lambda i: (0, 0)),
            pl.BlockSpec((n, n2), lambda i: (0, 0)),
            pl.BlockSpec((1, n2), lambda i: (0, 0)),
        ],
        out_specs=pl.BlockSpec((bm, n2), lambda i: (i, 0)),
        out_shape=jax.ShapeDtypeStruct((m, n2), jnp.float32),
    )(h, *ablks, wh, wa, b.reshape(1, n), wd, bd.reshape(1, n2))


def kernel(node_feat_table, nodes_id, W_init, b_init, W_msg, b_msg,
           W_upd, b_upd, W_dec, b_dec, edge_index, edge_attr, root_ids):
    src = edge_index[0]
    dst = edge_index[1]
    wm_h = W_msg[:HID]
    wm_e = W_msg[HID:]
    wu_h = W_upd[:HID]
    wu_a = W_upd[HID:]
    wd = jnp.zeros((HID, 128), jnp.float32)
    wd = wd.at[:, 0].set(W_dec[:HID, 0]).at[:, 1].set(W_dec[HID:, 0])
    bd = jnp.zeros((128,), jnp.float32).at[0].set(b_dec[0])
    rid = jnp.stack([root_ids[:, 0], root_ids[:, 1]], axis=1).reshape(-1)

    ori = _pre_gather(node_feat_table, nodes_id)
    eblks = _edge_mm(edge_attr, wm_e, b_msg)
    h0, *p1blks = _init_mm(ori, W_init, b_init, wm_h)
    a1blks = _hop(p1blks, eblks, src, dst)
    h1, *p2blks = _upd_mm(h0, a1blks, wu_h, wu_a, b_upd, wm_h)
    a2blks = _hop(p2blks, eblks, src, dst)
    d = _upd_dec_mm(h1, a2blks, wu_h, wu_a, b_upd, wd, bd)
    out = _decode(d, rid)
    return out.reshape(-1, 1)


# speedup vs baseline: 3.2518x; 1.6176x over previous
import jax
import jax.numpy as jnp
from jax import lax
from jax.experimental import pallas as pl
from jax.experimental.pallas import tpu as pltpu
from jax.experimental.pallas import tpu_sc as plsc

N_NODES = 10000
N_EDGES = 160000
HID = 512
CB = 128
NCB = HID // CB
NC = 2
NS = 16
EPT = N_EDGES // NS
EB = 80
NB = EPT // EB
RPT = 624
ZR = 48
HSL = CB // 16

_MESH = plsc.VectorSubcoreMesh(core_axis_name="c", subcore_axis_name="s")
_SC_PARAMS = pltpu.CompilerParams(needs_layout_passes=False)


def _pre_body(table, nid, ori, nidv, rows, sem):
    c = lax.axis_index("c")
    s = lax.axis_index("s")
    wid = s * NC + c
    start = jnp.minimum(wid * 320, N_NODES - 320)
    for b in range(5):
        off = start + b * 64
        pltpu.sync_copy(nid.at[pl.ds(off, 64)], nidv)
        pltpu.async_copy(table.at[nidv], rows, sem).wait()
        pltpu.sync_copy(rows, ori.at[pl.ds(off, 64)])


def _pre_gather(table, nid):
    k = pl.kernel(
        _pre_body,
        out_type=jax.ShapeDtypeStruct((N_NODES, 256), jnp.float32),
        mesh=_MESH,
        compiler_params=_SC_PARAMS,
        scratch_types=[
            pltpu.VMEM((64,), jnp.int32),
            pltpu.VMEM((64, 256), jnp.float32),
            pltpu.SemaphoreType.DMA,
        ],
    )
    return k(table, nid)


def _hop_body(p0, p1, p2, p3, e0, e1, e2, e3, src_hbm, dst_hbm,
              a0, a1, a2, a3,
              shared,
              prow0, prow1, erow0, erow1, srcv0, srcv1,
              dst0, dst1, dstS0, dstS1, zbuf,
              semC0, semC1, semG0, semG1, semS0, semS1):
    c = lax.axis_index("c")
    s = lax.axis_index("s")
    zero16 = jnp.zeros((16,), jnp.float32)

    def zb(r, z):
        for k in range(HSL):
            zbuf[r, pl.ds(k * 16, 16)] = zero16
        return z

    lax.fori_loop(0, ZR, zb, 0)
    ebase = s * EPT
    slots = (
        (prow0, erow0, srcv0, dst0, dstS0, semC0, semG0, semS0),
        (prow1, erow1, srcv1, dst1, dstS1, semC1, semG1, semS1),
    )

    def do_block(pq, eq, aq):
        def cp_pairs(b, slot):
            off = ebase + b * EB
            return (
                (src_hbm.at[pl.ds(off, EB)], slot[2]),
                (dst_hbm.at[pl.ds(off, EB)], slot[3].at[0]),
                (eq.at[pl.ds(off, EB)], slot[1]),
            )

        def issue_cp(b, slot):
            for sref, dref in cp_pairs(b, slot):
                pltpu.async_copy(sref, dref, slot[5])

        def wait_cp(b, slot):
            for sref, dref in cp_pairs(b, slot):
                pltpu.make_async_copy(sref, dref, slot[5]).wait()

        def issue_gather(slot):
            pltpu.async_copy(pq.at[slot[2]], slot[0], slot[6])

        def wait_gather(slot):
            pltpu.make_async_copy(pq.at[slot[2]], slot[0], slot[6]).wait()

        def issue_scatter(slot):
            pltpu.async_copy(slot[0], shared.at[slot[4].at[0]], slot[7],
                             add=True)

        def wait_scatter(slot):
            pltpu.make_async_copy(
                slot[0], shared.at[slot[4].at[0]], slot[7]).wait()

        def compute(slot):
            prow, erow, _, dst2d, dstS = slot[:5]
            for k in range(EB // 16):
                sl = pl.ds(k * 16, 16)
                dstS[0, sl] = dst2d[0, sl]

            def rfn(r, z2):
                for k in range(HSL):
                    sl = pl.ds(k * 16, 16)
                    prow[r, sl] = jnp.maximum(prow[r, sl] + erow[r, sl], 0.0)
                return z2

            lax.fori_loop(0, EB, rfn, 0)

        for j in range(13):
            pltpu.sync_copy(zbuf, shared.at[pl.ds(s * RPT + j * ZR, ZR)])

        @pl.when(s == NS - 1)
        def _():
            pltpu.sync_copy(zbuf.at[pl.ds(0, 16)],
                            shared.at[pl.ds(NS * RPT, 16)])

        plsc.subcore_barrier()

        issue_cp(0, slots[0])
        issue_cp(1, slots[1])
        wait_cp(0, slots[0])
        issue_gather(slots[0])

        def pair(g, z):
            b0 = 2 * g
            wait_cp(b0 + 1, slots[1])

            @pl.when(g > 0)
            def _():
                wait_scatter(slots[1])

            issue_gather(slots[1])
            wait_gather(slots[0])
            compute(slots[0])
            issue_scatter(slots[0])
            issue_cp(b0 + 2, slots[0])
            wait_cp(b0 + 2, slots[0])
            wait_scatter(slots[0])
            issue_gather(slots[0])
            wait_gather(slots[1])
            compute(slots[1])
            issue_scatter(slots[1])

            @pl.when(g < NB // 2 - 1)
            def _():
                issue_cp(b0 + 3, slots[1])

            return z

        lax.fori_loop(0, NB // 2, pair, 0)
        wait_scatter(slots[1])
        wait_gather(slots[0])
        compute(slots[0])
        issue_scatter(slots[0])
        wait_scatter(slots[0])
        plsc.subcore_barrier()

        pltpu.sync_copy(shared.at[pl.ds(s * RPT, RPT)],
                        aq.at[pl.ds(s * RPT, RPT)])

        @pl.when(s == NS - 1)
        def _():
            pltpu.sync_copy(shared.at[pl.ds(NS * RPT, 16)],
                            aq.at[pl.ds(NS * RPT, 16)])

        plsc.subcore_barrier()

    @pl.when(c == 0)
    def _():
        do_block(p0, e0, a0)
        do_block(p1, e1, a1)

    @pl.when(c == 1)
    def _():
        do_block(p2, e2, a2)
        do_block(p3, e3, a3)


def _hop(pblks, eblks, src, dst):
    k = pl.kernel(
        _hop_body,
        out_type=[jax.ShapeDtypeStruct((N_NODES, CB), jnp.float32)] * NCB,
        mesh=_MESH,
        compiler_params=_SC_PARAMS,
        scratch_types=[
            pltpu.VMEM_SHARED((N_NODES, CB), jnp.float32),
            pltpu.VMEM((EB, CB), jnp.float32),
            pltpu.VMEM((EB, CB), jnp.float32),
            pltpu.VMEM((EB, CB), jnp.float32),
            pltpu.VMEM((EB, CB), jnp.float32),
            pltpu.VMEM((EB,), jnp.int32),
            pltpu.VMEM((EB,), jnp.int32),
            pltpu.VMEM((1, EB), jnp.int32),
            pltpu.VMEM((1, EB), jnp.int32),
            pltpu.VMEM((1, EB), jnp.int32),
            pltpu.VMEM((1, EB), jnp.int32),
            pltpu.VMEM((ZR, CB), jnp.float32),
            pltpu.SemaphoreType.DMA,
            pltpu.SemaphoreType.DMA,
            pltpu.SemaphoreType.DMA,
            pltpu.SemaphoreType.DMA,
            pltpu.SemaphoreType.DMA,
            pltpu.SemaphoreType.DMA,
        ],
    )
    return k(*pblks, *eblks, src, dst)


def _dec_body(d_hbm, rid_hbm, out_hbm, ridv, rows, outv, sem):
    c = lax.axis_index("c")
    s = lax.axis_index("s")
    wid = s * NC + c
    pltpu.sync_copy(rid_hbm.at[pl.ds(wid * 64, 64)], ridv)
    pltpu.async_copy(d_hbm.at[ridv], rows, sem).wait()
    lane = lax.broadcasted_iota(jnp.int32, (16,), 0)
    col0 = jnp.zeros((16,), jnp.int32)
    for g in range(2):
        i0 = 2 * lane + g * 32
        a = plsc.load_gather(rows, [i0, col0])
        b = plsc.load_gather(rows, [i0 + 1, col0 + 1])
        outv[pl.ds(g * 16, 16)] = a + b
    pltpu.sync_copy(outv, out_hbm.at[pl.ds(wid * 32, 32)])


def _decode(d, rid):
    n_links = rid.shape[0] // 2
    k = pl.kernel(
        _dec_body,
        out_type=jax.ShapeDtypeStruct((n_links,), jnp.float32),
        mesh=_MESH,
        compiler_params=_SC_PARAMS,
        scratch_types=[
            pltpu.VMEM((64,), jnp.int32),
            pltpu.VMEM((64, 128), jnp.float32),
            pltpu.VMEM((32,), jnp.float32),
            pltpu.SemaphoreType.DMA,
        ],
    )
    return k(d, rid)


def _emm_kern(x_ref, w_ref, b_ref, o0, o1, o2, o3):
    o = (jnp.dot(x_ref[...], w_ref[...],
                 preferred_element_type=jnp.float32) + b_ref[...])
    for q, oq in enumerate((o0, o1, o2, o3)):
        oq[...] = o[:, q * CB:(q + 1) * CB]


def _edge_mm(x, w, b):
    m, kdim = x.shape
    n = w.shape[1]
    bm = 2000
    return pl.pallas_call(
        _emm_kern,
        grid=(m // bm,),
        in_specs=[
            pl.BlockSpec((bm, kdim), lambda i: (i, 0)),
            pl.BlockSpec((kdim, n), lambda i: (0, 0)),
            pl.BlockSpec((1, n), lambda i: (0, 0)),
        ],
        out_specs=[pl.BlockSpec((bm, CB), lambda i: (i, 0))] * NCB,
        out_shape=[jax.ShapeDtypeStruct((m, CB), jnp.float32)] * NCB,
    )(x, w, b.reshape(1, n))


def _init_kern(x_ref, wi_ref, bi_ref, wm_ref, h_ref, p0, p1, p2, p3):
    h = (jnp.dot(x_ref[...], wi_ref[...],
                 preferred_element_type=jnp.float32) + bi_ref[...])
    h_ref[...] = h
    for q, pq in enumerate((p0, p1, p2, p3)):
        pq[...] = jnp.dot(h, wm_ref[:, q * CB:(q + 1) * CB],
                          preferred_element_type=jnp.float32)


def _init_mm(x, wi, bi, wm):
    m, kdim = x.shape
    n = wi.shape[1]
    bm = 1000
    return pl.pallas_call(
        _init_kern,
        grid=(m // bm,),
        in_specs=[
            pl.BlockSpec((bm, kdim), lambda i: (i, 0)),
            pl.BlockSpec((kdim, n), lambda i: (0, 0)),
            pl.BlockSpec((1, n), lambda i: (0, 0)),
            pl.BlockSpec((n, n), lambda i: (0, 0)),
        ],
        out_specs=[pl.BlockSpec((bm, n), lambda i: (i, 0))]
        + [pl.BlockSpec((bm, CB), lambda i: (i, 0))] * NCB,
        out_shape=[jax.ShapeDtypeStruct((m, n), jnp.float32)]
        + [jax.ShapeDtypeStruct((m, CB), jnp.float32)] * NCB,
    )(x, wi, bi.reshape(1, n), wm)


def _upd_kern(h_ref, a0, a1, a2, a3, wh_ref, wa_ref, b_ref, wn_ref,
              h2_ref, p0, p1, p2, p3):
    y = (jnp.dot(h_ref[...], wh_ref[...], preferred_element_type=jnp.float32)
         + b_ref[...])
    for q, aq in enumerate((a0, a1, a2, a3)):
        y = y + jnp.dot(aq[...], wa_ref[q * CB:(q + 1) * CB, :],
                        preferred_element_type=jnp.float32)
    y = jnp.maximum(y, 0.0)
    h2_ref[...] = y
    for q, pq in enumerate((p0, p1, p2, p3)):
        pq[...] = jnp.dot(y, wn_ref[:, q * CB:(q + 1) * CB],
                          preferred_element_type=jnp.float32)


def _upd_mm(h, ablks, wh, wa, b, wn):
    m, n = h.shape
    bm = 1000
    return pl.pallas_call(
        _upd_kern,
        grid=(m // bm,),
        in_specs=[pl.BlockSpec((bm, n), lambda i: (i, 0))]
        + [pl.BlockSpec((bm, CB), lambda i: (i, 0))] * NCB
        + [
            pl.BlockSpec((n, n), lambda i: (0, 0)),
            pl.BlockSpec((n, n), lambda i: (0, 0)),
            pl.BlockSpec((1, n), lambda i: (0, 0)),
            pl.BlockSpec((n, n), lambda i: (0, 0)),
        ],
        out_specs=[pl.BlockSpec((bm, n), lambda i: (i, 0))]
        + [pl.BlockSpec((bm, CB), lambda i: (i, 0))] * NCB,
        out_shape=[jax.ShapeDtypeStruct((m, n), jnp.float32)]
        + [jax.ShapeDtypeStruct((m, CB), jnp.float32)] * NCB,
    )(h, *ablks, wh, wa, b.reshape(1, n), wn)


def _upd_dec_kern(h_ref, a0, a1, a2, a3, wh_ref, wa_ref, b_ref,
                  wd_ref, bd_ref, d_ref):
    y = (jnp.dot(h_ref[...], wh_ref[...], preferred_element_type=jnp.float32)
         + b_ref[...])
    for q, aq in enumerate((a0, a1, a2, a3)):
        y = y + jnp.dot(aq[...], wa_ref[q * CB:(q + 1) * CB, :],
                        preferred_element_type=jnp.float32)
    y = jnp.maximum(y, 0.0)
    d_ref[...] = (jnp.dot(y, wd_ref[...], preferred_element_type=jnp.float32)
                  + bd_ref[...])


def _upd_dec_mm(h, ablks, wh, wa, b, wd, bd):
    m, n = h.shape
    n2 = wd.shape[1]
    bm = 1000
    return pl.pallas_call(
        _upd_dec_kern,
        grid=(m // bm,),
        in_specs=[pl.BlockSpec((bm, n), lambda i: (i, 0))]
        + [pl.BlockSpec((bm, CB), lambda i: (i, 0))] * NCB
        + [
            pl.BlockSpec((n, n), lambda i: (0, 0)),
            pl.BlockSpec((n, n), lambda i: (0, 0)),
            pl.BlockSpec((1, n), lambda i: (0, 0)),
            pl.BlockSpec((n, n2), lambda i: (0, 0)),
            pl.BlockSpec((1, n2), lambda i: (0, 0)),
        ],
        out_specs=pl.BlockSpec((bm, n2), lambda i: (i, 0)),
        out_shape=jax.ShapeDtypeStruct((m, n2), jnp.float32),
    )(h, *ablks, wh, wa, b.reshape(1, n), wd, bd.reshape(1, n2))


def kernel(node_feat_table, nodes_id, W_init, b_init, W_msg, b_msg,
           W_upd, b_upd, W_dec, b_dec, edge_index, edge_attr, root_ids):
    src = edge_index[0]
    dst = edge_index[1]
    wm_h = W_msg[:HID]
    wm_e = W_msg[HID:]
    wu_h = W_upd[:HID]
    wu_a = W_upd[HID:]
    wd = jnp.zeros((HID, 128), jnp.float32)
    wd = wd.at[:, 0].set(W_dec[:HID, 0]).at[:, 1].set(W_dec[HID:, 0])
    bd = jnp.zeros((128,), jnp.float32).at[0].set(b_dec[0])
    rid = jnp.stack([root_ids[:, 0], root_ids[:, 1]], axis=1).reshape(-1)

    ori = _pre_gather(node_feat_table, nodes_id)
    eblks = _edge_mm(edge_attr, wm_e, b_msg)
    h0, *p1blks = _init_mm(ori, W_init, b_init, wm_h)
    a1blks = _hop(p1blks, eblks, src, dst)
    h1, *p2blks = _upd_mm(h0, a1blks, wu_h, wu_a, b_upd, wm_h)
    a2blks = _hop(p2blks, eblks, src, dst)
    d = _upd_dec_mm(h1, a2blks, wu_h, wu_a, b_upd, wd, bd)
    out = _decode(d, rid)
    return out.reshape(-1, 1)
